# baseline (speedup 1.0000x reference)
import functools
import math

import jax
import jax.numpy as jnp
from jax import lax
from jax.experimental import pallas as pl
from jax.experimental.pallas import tpu as pltpu
from jax.experimental.pallas import tpu_sc as plsc

VOCAB = 100000
EMB = 128
SCALE = math.sqrt(EMB)

NC = 2
NS = 16
NW = NC * NS

G = 128
NBUF = 5
LANES = 16


def _make_sc_gather(n_flat):
    b_per_w = n_flat // NW
    ng = b_per_w // G
    mesh = plsc.VectorSubcoreMesh(core_axis_name="c", subcore_axis_name="s")

    @functools.partial(
        pl.kernel,
        mesh=mesh,
        out_type=jax.ShapeDtypeStruct((n_flat, EMB), jnp.float32),
        scratch_types=[
            pltpu.VMEM((ng, G), jnp.int32),
            pltpu.VMEM((NBUF, G, EMB), jnp.float32),
            pltpu.SemaphoreType.DMA,
            pltpu.SemaphoreType.DMA,
        ],
    )
    def sc_gather(table_hbm, idx_hbm, out_hbm, idx_v, buf_v, gsem, osem):
        wid = lax.axis_index("s") * NC + lax.axis_index("c")
        base = wid * b_per_w
        pltpu.sync_copy(idx_hbm.at[wid], idx_v)

        for b in range(NBUF):
            pltpu.async_copy(table_hbm.at[idx_v.at[b]], buf_v.at[b], gsem)

        def scale_rows(b, lo, hi):
            def row(r, _):
                for c in range(EMB // LANES):
                    sl = pl.ds(c * LANES, LANES)
                    buf_v[b, r, sl] = buf_v[b, r, sl] * SCALE
                return ()

            lax.fori_loop(lo, hi, row, (), unroll=2)

        def step(jj, _):
            j0 = jj * NBUF
            for b in range(NBUF):
                j = j0 + b
                pltpu.make_async_copy(
                    table_hbm.at[idx_v.at[0]], buf_v.at[b], gsem).wait()
                half = G // 2
                scale_rows(b, 0, half)
                pltpu.async_copy(
                    buf_v.at[b, pl.ds(0, half)],
                    out_hbm.at[pl.ds(base + j * G, half)], osem)
                scale_rows(b, half, G)
                pltpu.async_copy(
                    buf_v.at[b, pl.ds(half, half)],
                    out_hbm.at[pl.ds(base + j * G + half, half)], osem)
                nj = j + 3
                b2 = (b + 3) % NBUF

                @pl.when((nj >= NBUF) & (nj < ng))
                def _():
                    pltpu.make_async_copy(
                        buf_v.at[b2], out_hbm.at[pl.ds(base, G)], osem).wait()
                    pltpu.async_copy(
                        table_hbm.at[idx_v.at[nj]], buf_v.at[b2], gsem)
            return ()

        lax.fori_loop(0, ng // NBUF, step, (), unroll=False)

        for b in range(NBUF):
            pltpu.make_async_copy(
                buf_v.at[b], out_hbm.at[pl.ds(base, G)], osem).wait()

    return sc_gather


def kernel(tokens, table):
    b, l = tokens.shape
    n_flat = b * l
    b_per_w = n_flat // NW
    ng = b_per_w // G
    idx = tokens.reshape(NW, ng, G).astype(jnp.int32)
    out = _make_sc_gather(n_flat)(table, idx)
    return out.reshape(b, l, EMB)


# speedup vs baseline: 1.0077x; 1.0077x over previous
import functools
import math

import jax
import jax.numpy as jnp
from jax import lax
from jax.experimental import pallas as pl
from jax.experimental.pallas import tpu as pltpu
from jax.experimental.pallas import tpu_sc as plsc

VOCAB = 100000
EMB = 128
SCALE = math.sqrt(EMB)

NC = 2
NS = 16
NW = NC * NS

G = 128
NBUF = 5
LANES = 16


def _make_sc_gather(n_flat):
    b_per_w = n_flat // NW
    ng = b_per_w // G
    mesh = plsc.VectorSubcoreMesh(core_axis_name="c", subcore_axis_name="s")

    @functools.partial(
        pl.kernel,
        mesh=mesh,
        out_type=jax.ShapeDtypeStruct((n_flat, EMB), jnp.float32),
        scratch_types=[
            pltpu.VMEM((ng, G), jnp.int32),
            pltpu.VMEM((NBUF, G, EMB), jnp.float32),
            pltpu.SemaphoreType.DMA,
            pltpu.SemaphoreType.DMA,
        ],
    )
    def sc_gather(table_hbm, idx_hbm, out_hbm, idx_v, buf_v, gsem, osem):
        wid = lax.axis_index("s") * NC + lax.axis_index("c")
        base = wid * b_per_w
        pltpu.sync_copy(idx_hbm.at[wid], idx_v)

        for b in range(NBUF):
            pltpu.async_copy(table_hbm.at[idx_v.at[b]], buf_v.at[b], gsem)

        def scale_rows(b, lo, hi):
            def row(r, _):
                for c in range(EMB // LANES):
                    sl = pl.ds(c * LANES, LANES)
                    buf_v[b, r, sl] = buf_v[b, r, sl] * SCALE
                return ()

            lax.fori_loop(lo, hi, row, (), unroll=2)

        def step(jj, _):
            j0 = jj * NBUF
            for b in range(NBUF):
                j = j0 + b
                pltpu.make_async_copy(
                    table_hbm.at[idx_v.at[0]], buf_v.at[b], gsem).wait()
                half = G // 2
                scale_rows(b, 0, half)
                pltpu.async_copy(
                    buf_v.at[b, pl.ds(0, half)],
                    out_hbm.at[pl.ds(base + j * G, half)], osem)
                scale_rows(b, half, G)
                pltpu.async_copy(
                    buf_v.at[b, pl.ds(half, half)],
                    out_hbm.at[pl.ds(base + j * G + half, half)], osem)
                nj = j + 2
                b2 = (b + 2) % NBUF

                @pl.when((nj >= NBUF) & (nj < ng))
                def _():
                    pltpu.make_async_copy(
                        buf_v.at[b2], out_hbm.at[pl.ds(base, G)], osem).wait()
                    pltpu.async_copy(
                        table_hbm.at[idx_v.at[nj]], buf_v.at[b2], gsem)
            return ()

        lax.fori_loop(0, ng // NBUF, step, (), unroll=False)

        for b in range(NBUF):
            pltpu.make_async_copy(
                buf_v.at[b], out_hbm.at[pl.ds(base, G)], osem).wait()

    return sc_gather


def kernel(tokens, table):
    b, l = tokens.shape
    n_flat = b * l
    b_per_w = n_flat // NW
    ng = b_per_w // G
    idx = tokens.reshape(NW, ng, G).astype(jnp.int32)
    out = _make_sc_gather(n_flat)(table, idx)
    return out.reshape(b, l, EMB)
